# Initial kernel scaffold; baseline (speedup 1.0000x reference)
import jax
import jax.numpy as jnp
from jax.experimental import pallas as pl


def kernel(x, tables, W, b):
    raise NotImplementedError("write your pallas kernel here")


# speedup vs baseline: 7.9837x; 7.9837x over previous
import functools

import jax
import jax.numpy as jnp
from jax import lax
from jax.experimental import pallas as pl
from jax.experimental.pallas import tpu as pltpu
from jax.experimental.pallas import tpu_sc as plsc

_N_CATS = 26
_VOCAB = 100000
_N_EMBED = 16
_N_NUM = 13
_BATCH = 16384

_NC = 2
_NS = 16
_NW = _NC * _NS
_SPW = _BATCH // _NW
_CHUNK = 128
_NCHUNK = _SPW // _CHUNK
_IPC = _CHUNK * _N_CATS
_IDX_W = 128
_IDX_R = _IPC // _IDX_W


def _sc_gather_mac(xi_flat, tab_flat, w_mat, offs):
    mesh = plsc.VectorSubcoreMesh(core_axis_name="c", subcore_axis_name="s")

    @functools.partial(
        pl.kernel,
        out_type=jax.ShapeDtypeStruct((_BATCH, _N_EMBED), jnp.float32),
        mesh=mesh,
        scratch_types=[
            pltpu.VMEM((_IPC,), jnp.int32),
            pltpu.VMEM((_IPC,), jnp.int32),
            pltpu.VMEM((_IPC,), jnp.int32),
            pltpu.VMEM((_IPC, _N_EMBED), jnp.float32),
            pltpu.VMEM((_CHUNK, _N_EMBED), jnp.float32),
            pltpu.VMEM((_N_CATS, _N_EMBED), jnp.float32),
            pltpu.SemaphoreType.DMA,
        ],
        compiler_params=pltpu.CompilerParams(use_tc_tiling_on_sc=False),
    )
    def k(xi_hbm, tab_hbm, w_hbm, offs_hbm, out_hbm,
          idxraw_v, idx_v, offs_v, rows_v, acc_v, w_v, sem):
        wid = lax.axis_index("s") * _NC + lax.axis_index("c")
        base = wid * _SPW

        pltpu.sync_copy(w_hbm, w_v)
        pltpu.sync_copy(offs_hbm, offs_v)
        wvecs = [w_v[i] for i in range(_N_CATS)]

        @pl.loop(0, _NCHUNK)
        def _chunk(c):
            cbase = base + c * _CHUNK
            i0 = pl.multiple_of(cbase * _N_CATS, _IPC)

            pltpu.sync_copy(xi_hbm.at[pl.ds(i0, _IPC)], idxraw_v)

            @pl.loop(0, _IPC // 16)
            def _off(v):
                sl = pl.ds(pl.multiple_of(v * 16, 16), 16)
                idx_v[sl] = idxraw_v[sl] + offs_v[sl]

            copies = [
                pltpu.async_copy(
                    tab_hbm.at[idx_v.at[pl.ds(j * _IDX_W, _IDX_W)]],
                    rows_v.at[pl.ds(j * _IDX_W, _IDX_W)],
                    sem,
                )
                for j in range(_IDX_R)
            ]
            for cp in copies:
                cp.wait()

            @pl.loop(0, _CHUNK)
            def _mac(s):
                r0 = s * _N_CATS
                acc = rows_v[r0] * wvecs[0]
                for i in range(1, _N_CATS):
                    acc = acc + rows_v[r0 + i] * wvecs[i]
                acc_v[s] = acc

            pltpu.sync_copy(
                acc_v, out_hbm.at[pl.ds(pl.multiple_of(cbase, _CHUNK), _CHUNK)])

    return k(xi_flat, tab_flat, w_mat, offs)


_BLK = 2048


def _finish(acc, x, w_num, bias):
    def body(acc_ref, x_ref, wn_ref, b_ref, o_ref):
        e = jnp.sum(acc_ref[...], axis=1, keepdims=True)
        xn = x_ref[...][:, _N_CATS:]
        nsum = jnp.sum(xn * wn_ref[...], axis=1, keepdims=True)
        o_ref[...] = e + nsum + b_ref[0, 0]

    return pl.pallas_call(
        body,
        grid=(_BATCH // _BLK,),
        in_specs=[
            pl.BlockSpec((_BLK, _N_EMBED), lambda i: (i, 0)),
            pl.BlockSpec((_BLK, _N_CATS + _N_NUM), lambda i: (i, 0)),
            pl.BlockSpec((1, _N_NUM), lambda i: (0, 0)),
            pl.BlockSpec((1, 1), lambda i: (0, 0)),
        ],
        out_specs=pl.BlockSpec((_BLK, 1), lambda i: (i, 0)),
        out_shape=jax.ShapeDtypeStruct((_BATCH, 1), jnp.float32),
    )(acc, x, w_num, bias)


def kernel(x, tables, W, b):
    xi = x[:, :_N_CATS].astype(jnp.int32)
    xi_flat = xi.reshape(_BATCH * _N_CATS)
    tab_flat = tables.reshape(_N_CATS * _VOCAB, _N_EMBED)
    w_mat = W[0, : _N_CATS * _N_EMBED].reshape(_N_CATS, _N_EMBED)
    w_num = W[0, _N_CATS * _N_EMBED:].reshape(1, _N_NUM)
    bias = b.reshape(1, 1)
    offs = (jnp.arange(_IPC, dtype=jnp.int32) % _N_CATS) * _VOCAB

    acc = _sc_gather_mac(xi_flat, tab_flat, w_mat, offs)
    return _finish(acc, x, w_num, bias)


# speedup vs baseline: 40.7370x; 5.1025x over previous
import dataclasses
import functools

import jax
import jax.numpy as jnp
from jax import lax
from jax.experimental import pallas as pl
from jax.experimental.pallas import tpu as pltpu
from jax.experimental.pallas import tpu_sc as plsc

_N_CATS = 26
_VOCAB = 100000
_VPAD = 100352
_N_EMBED = 16
_N_NUM = 13
_BATCH = 16384

_NC = 2
_NS = 16
_NW = _NC * _NS
_SPW = _BATCH // _NW
_CHUNK = 128
_NCHUNK = _SPW // _CHUNK
_IPC = _CHUNK * _N_CATS
_IDX_W = 128
_IDX_R = _IPC // _IDX_W

_PBLK = 14336


def _project(tab_t, w_mat):
    def body(tab_ref, w_ref, p_ref):
        p_ref[...] = jnp.sum(tab_ref[0] * w_ref[0], axis=0)

    return pl.pallas_call(
        body,
        grid=(_N_CATS, _VPAD // _PBLK),
        in_specs=[
            pl.BlockSpec((1, _N_EMBED, _PBLK), lambda t, j: (t, 0, j)),
            pl.BlockSpec((1, _N_EMBED, 1), lambda t, j: (t, 0, 0)),
        ],
        out_specs=pl.BlockSpec((_PBLK,), lambda t, j: (t * (_VPAD // _PBLK) + j,)),
        out_shape=jax.ShapeDtypeStruct((_N_CATS * _VPAD,), jnp.float32),
    )(tab_t, w_mat[:, :, None])


def _sc_compiler_params():
    cp = pltpu.CompilerParams(use_tc_tiling_on_sc=False)
    if "needs_layout_passes" in pltpu.CompilerParams.__dataclass_fields__:
        cp = dataclasses.replace(cp, needs_layout_passes=False)
    return cp


def _sc_gather_sum(xi_flat, p_flat, offs):
    mesh = plsc.VectorSubcoreMesh(core_axis_name="c", subcore_axis_name="s")

    @functools.partial(
        pl.kernel,
        out_type=jax.ShapeDtypeStruct((_BATCH,), jnp.float32),
        mesh=mesh,
        scratch_types=[
            pltpu.VMEM((_IPC,), jnp.int32),
            pltpu.VMEM((_IPC,), jnp.int32),
            pltpu.VMEM((_IPC,), jnp.int32),
            pltpu.VMEM((_IPC,), jnp.float32),
            pltpu.VMEM((_CHUNK,), jnp.float32),
            pltpu.SemaphoreType.DMA,
        ],
        compiler_params=_sc_compiler_params(),
    )
    def k(xi_hbm, p_hbm, offs_hbm, out_hbm,
          idxraw_v, idx_v, offs_v, vals_v, sum_v, sem):
        wid = lax.axis_index("s") * _NC + lax.axis_index("c")
        base = wid * _SPW

        pltpu.sync_copy(offs_hbm, offs_v)
        lane26 = lax.iota(jnp.int32, 16) * _N_CATS

        @pl.loop(0, _NCHUNK)
        def _chunk(c):
            cbase = base + c * _CHUNK
            i0 = pl.multiple_of(cbase * _N_CATS, _IPC)

            pltpu.sync_copy(xi_hbm.at[pl.ds(i0, _IPC)], idxraw_v)

            @pl.loop(0, _IPC // 16)
            def _off(v):
                sl = pl.ds(pl.multiple_of(v * 16, 16), 16)
                idx_v[sl] = idxraw_v[sl] + offs_v[sl]

            copies = [
                pltpu.async_copy(
                    p_hbm.at[idx_v.at[pl.ds(j * _IDX_W, _IDX_W)]],
                    vals_v.at[pl.ds(j * _IDX_W, _IDX_W)],
                    sem,
                )
                for j in range(_IDX_R)
            ]
            for cp in copies:
                cp.wait()

            for g in range(_CHUNK // 16):
                acc = plsc.load_gather(vals_v, [lane26 + (g * 16 * _N_CATS)])
                for j in range(1, _N_CATS):
                    acc = acc + plsc.load_gather(
                        vals_v, [lane26 + (g * 16 * _N_CATS + j)])
                sum_v[pl.ds(g * 16, 16)] = acc

            pltpu.sync_copy(
                sum_v, out_hbm.at[pl.ds(pl.multiple_of(cbase, _CHUNK), _CHUNK)])

    return k(xi_flat, p_flat, offs)


_BLK = 2048


def _finish(emb, x, w_num, bias):
    def body(emb_ref, x_ref, wn_ref, b_ref, o_ref):
        xn = x_ref[...][:, _N_CATS:]
        nsum = jnp.sum(xn * wn_ref[...], axis=1)
        o_ref[...] = emb_ref[...] + nsum + b_ref[0, 0]

    return pl.pallas_call(
        body,
        grid=(_BATCH // _BLK,),
        in_specs=[
            pl.BlockSpec((_BLK,), lambda i: (i,)),
            pl.BlockSpec((_BLK, _N_CATS + _N_NUM), lambda i: (i, 0)),
            pl.BlockSpec((1, _N_NUM), lambda i: (0, 0)),
            pl.BlockSpec((1, 1), lambda i: (0, 0)),
        ],
        out_specs=pl.BlockSpec((_BLK,), lambda i: (i,)),
        out_shape=jax.ShapeDtypeStruct((_BATCH,), jnp.float32),
    )(emb, x, w_num, bias)


def kernel(x, tables, W, b):
    xi = x[:, :_N_CATS].astype(jnp.int32)
    xi_flat = xi.reshape(_BATCH * _N_CATS)
    tab_t = jnp.transpose(tables, (0, 2, 1))
    w_mat = W[0, : _N_CATS * _N_EMBED].reshape(_N_CATS, _N_EMBED)
    w_num = W[0, _N_CATS * _N_EMBED:].reshape(1, _N_NUM)
    bias = b.reshape(1, 1)
    offs = (jnp.arange(_IPC, dtype=jnp.int32) % _N_CATS) * _VPAD

    p_flat = _project(tab_t, w_mat)
    emb = _sc_gather_sum(xi_flat, p_flat, offs)
    return _finish(emb, x, w_num, bias).reshape(_BATCH, 1)


# speedup vs baseline: 71.9463x; 1.7661x over previous
import dataclasses
import functools

import jax
import jax.numpy as jnp
from jax import lax
from jax.experimental import pallas as pl
from jax.experimental.pallas import tpu as pltpu
from jax.experimental.pallas import tpu_sc as plsc

_N_CATS = 26
_VOCAB = 100000
_VPAD = 100352
_N_EMBED = 16
_N_NUM = 13
_N_FEAT = _N_CATS + _N_NUM
_BATCH = 16384

_NC = 2
_NS = 16
_NW = _NC * _NS
_SPW = _BATCH // _NW
_CHUNK = 128
_NCHUNK = _SPW // _CHUNK

_PBLK = 50176


def _indices(x_t):
    def body(x_ref, o_ref):
        a = pl.program_id(0)
        for r in range(8):
            o_ref[pl.ds(r * _BATCH, _BATCH)] = (
                x_ref[r].astype(jnp.int32) + (a * 8 + r) * _VPAD)

    return pl.pallas_call(
        body,
        grid=(_N_FEAT // 8,),
        in_specs=[pl.BlockSpec((8, _BATCH), lambda a: (a, 0))],
        out_specs=pl.BlockSpec((8 * _BATCH,), lambda a: (a,)),
        out_shape=jax.ShapeDtypeStruct((32 * _BATCH,), jnp.int32),
    )(x_t)


def _project(tab_t, w_mat):
    def body(tab_ref, w_ref, p_ref):
        p_ref[...] = jnp.sum(tab_ref[0] * w_ref[0], axis=0)

    return pl.pallas_call(
        body,
        grid=(_N_CATS, _VPAD // _PBLK),
        in_specs=[
            pl.BlockSpec((1, _N_EMBED, _PBLK), lambda t, j: (t, 0, j)),
            pl.BlockSpec((1, _N_EMBED, 1), lambda t, j: (t, 0, 0)),
        ],
        out_specs=pl.BlockSpec((_PBLK,), lambda t, j: (t * (_VPAD // _PBLK) + j,)),
        out_shape=jax.ShapeDtypeStruct((_N_CATS * _VPAD,), jnp.float32),
    )(tab_t, w_mat[:, :, None])


def _sc_compiler_params():
    cp = pltpu.CompilerParams(use_tc_tiling_on_sc=False)
    if "needs_layout_passes" in pltpu.CompilerParams.__dataclass_fields__:
        cp = dataclasses.replace(cp, needs_layout_passes=False)
    return cp


def _sc_gather_sum(xi_flat, p_flat):
    mesh = plsc.VectorSubcoreMesh(core_axis_name="c", subcore_axis_name="s")

    @functools.partial(
        pl.kernel,
        out_type=jax.ShapeDtypeStruct((_BATCH,), jnp.float32),
        mesh=mesh,
        scratch_types=[
            pltpu.VMEM((_N_CATS * _SPW,), jnp.int32),
            pltpu.VMEM((_N_CATS * _CHUNK,), jnp.float32),
            pltpu.VMEM((_CHUNK,), jnp.float32),
            pltpu.SemaphoreType.DMA,
            pltpu.SemaphoreType.DMA,
        ],
        compiler_params=_sc_compiler_params(),
    )
    def k(xi_hbm, p_hbm, out_hbm, idx_v, vals_v, sum_v, isem, gsem):
        wid = lax.axis_index("s") * _NC + lax.axis_index("c")
        base = wid * _SPW

        idx_copies = [
            pltpu.async_copy(
                xi_hbm.at[pl.ds(pl.multiple_of(j * _BATCH + base, _SPW), _SPW)],
                idx_v.at[pl.ds(j * _SPW, _SPW)],
                isem,
            )
            for j in range(_N_CATS)
        ]
        for cp in idx_copies:
            cp.wait()

        @pl.loop(0, _NCHUNK)
        def _chunk(c):
            coff = c * _CHUNK

            gathers = [
                pltpu.async_copy(
                    p_hbm.at[idx_v.at[pl.ds(
                        pl.multiple_of(j * _SPW + coff, _CHUNK), _CHUNK)]],
                    vals_v.at[pl.ds(j * _CHUNK, _CHUNK)],
                    gsem,
                )
                for j in range(_N_CATS)
            ]
            for cp in gathers:
                cp.wait()

            for g in range(_CHUNK // 16):
                acc = vals_v[pl.ds(g * 16, 16)]
                for j in range(1, _N_CATS):
                    acc = acc + vals_v[pl.ds(j * _CHUNK + g * 16, 16)]
                sum_v[pl.ds(g * 16, 16)] = acc

            pltpu.sync_copy(
                sum_v,
                out_hbm.at[pl.ds(pl.multiple_of(base + coff, _CHUNK), _CHUNK)])

    return k(xi_flat, p_flat)


_BLK = 2048


def _finish(emb, x_t, w_full_t, bias):
    def body(emb_ref, x_ref, wn_ref, b_ref, o_ref):
        nsum = jnp.sum(x_ref[...] * wn_ref[...], axis=0)
        o_ref[...] = emb_ref[...] + nsum + b_ref[0, 0]

    return pl.pallas_call(
        body,
        grid=(_BATCH // _BLK,),
        in_specs=[
            pl.BlockSpec((_BLK,), lambda i: (i,)),
            pl.BlockSpec((_N_FEAT, _BLK), lambda i: (0, i)),
            pl.BlockSpec((_N_FEAT, 1), lambda i: (0, 0)),
            pl.BlockSpec((1, 1), lambda i: (0, 0)),
        ],
        out_specs=pl.BlockSpec((_BLK,), lambda i: (i,)),
        out_shape=jax.ShapeDtypeStruct((_BATCH,), jnp.float32),
    )(emb, x_t, w_full_t, bias)


def kernel(x, tables, W, b):
    x_t = jnp.transpose(x, (1, 0))
    tab_t = jnp.transpose(tables, (0, 2, 1))
    w_mat = W[0, : _N_CATS * _N_EMBED].reshape(_N_CATS, _N_EMBED)
    w_full_t = jnp.concatenate(
        [jnp.zeros((_N_CATS,), jnp.float32), W[0, _N_CATS * _N_EMBED:]]
    ).reshape(_N_FEAT, 1)
    bias = b.reshape(1, 1)

    xi_flat = _indices(x_t)
    p_flat = _project(tab_t, w_mat)
    emb = _sc_gather_sum(xi_flat, p_flat)
    return _finish(emb, x_t, w_full_t, bias).reshape(_BATCH, 1)


# speedup vs baseline: 80.9832x; 1.1256x over previous
import dataclasses
import functools

import jax
import jax.numpy as jnp
from jax import lax
from jax.experimental import pallas as pl
from jax.experimental.pallas import tpu as pltpu
from jax.experimental.pallas import tpu_sc as plsc

_N_CATS = 26
_VOCAB = 100000
_VPAD = 100352
_N_EMBED = 16
_N_NUM = 13
_N_FEAT = _N_CATS + _N_NUM
_BATCH = 16384

_NC = 2
_NS = 16
_NW = _NC * _NS
_SPW = _BATCH // _NW
_CHUNK = 128
_NCHUNK = _SPW // _CHUNK

_PBLK = 100352


def _indices(x_t):
    def body(x_ref, o_ref):
        a = pl.program_id(0)
        for r in range(8):
            o_ref[pl.ds(r * _BATCH, _BATCH)] = (
                x_ref[r].astype(jnp.int32) + (a * 8 + r) * _VPAD)

    return pl.pallas_call(
        body,
        grid=(_N_FEAT // 8,),
        in_specs=[pl.BlockSpec((8, _BATCH), lambda a: (a, 0))],
        out_specs=pl.BlockSpec((8 * _BATCH,), lambda a: (a,)),
        out_shape=jax.ShapeDtypeStruct((32 * _BATCH,), jnp.int32),
    )(x_t)


def _project(tab_t, w_mat):
    def body(tab_ref, w_ref, p_ref):
        p_ref[...] = jnp.sum(tab_ref[0] * w_ref[0], axis=0)

    return pl.pallas_call(
        body,
        grid=(_N_CATS, _VPAD // _PBLK),
        in_specs=[
            pl.BlockSpec((1, _N_EMBED, _PBLK), lambda t, j: (t, 0, j)),
            pl.BlockSpec((1, _N_EMBED, 1), lambda t, j: (t, 0, 0)),
        ],
        out_specs=pl.BlockSpec((_PBLK,), lambda t, j: (t * (_VPAD // _PBLK) + j,)),
        out_shape=jax.ShapeDtypeStruct((_N_CATS * _VPAD,), jnp.float32),
    )(tab_t, w_mat[:, :, None])


def _sc_compiler_params():
    cp = pltpu.CompilerParams(use_tc_tiling_on_sc=False)
    if "needs_layout_passes" in pltpu.CompilerParams.__dataclass_fields__:
        cp = dataclasses.replace(cp, needs_layout_passes=False)
    return cp


def _sc_gather_sum(xi_flat, p_flat):
    mesh = plsc.VectorSubcoreMesh(core_axis_name="c", subcore_axis_name="s")

    @functools.partial(
        pl.kernel,
        out_type=jax.ShapeDtypeStruct((_BATCH,), jnp.float32),
        mesh=mesh,
        scratch_types=[
            pltpu.VMEM((_N_CATS * _SPW,), jnp.int32),
            pltpu.VMEM((_N_CATS * _CHUNK,), jnp.float32),
            pltpu.VMEM((_CHUNK,), jnp.float32),
            pltpu.SemaphoreType.DMA,
            pltpu.SemaphoreType.DMA,
        ],
        compiler_params=_sc_compiler_params(),
    )
    def k(xi_hbm, p_hbm, out_hbm, idx_v, vals_v, sum_v, isem, gsem):
        wid = lax.axis_index("s") * _NC + lax.axis_index("c")
        base = wid * _SPW

        idx_copies = [
            pltpu.async_copy(
                xi_hbm.at[pl.ds(pl.multiple_of(j * _BATCH + base, _SPW), _SPW)],
                idx_v.at[pl.ds(j * _SPW, _SPW)],
                isem,
            )
            for j in range(_N_CATS)
        ]
        for cp in idx_copies:
            cp.wait()

        @pl.loop(0, _NCHUNK)
        def _chunk(c):
            coff = c * _CHUNK

            gathers = [
                pltpu.async_copy(
                    p_hbm.at[idx_v.at[pl.ds(
                        pl.multiple_of(j * _SPW + coff, _CHUNK), _CHUNK)]],
                    vals_v.at[pl.ds(j * _CHUNK, _CHUNK)],
                    gsem,
                )
                for j in range(_N_CATS)
            ]
            for cp in gathers:
                cp.wait()

            for g in range(_CHUNK // 16):
                acc = vals_v[pl.ds(g * 16, 16)]
                for j in range(1, _N_CATS):
                    acc = acc + vals_v[pl.ds(j * _CHUNK + g * 16, 16)]
                sum_v[pl.ds(g * 16, 16)] = acc

            pltpu.sync_copy(
                sum_v,
                out_hbm.at[pl.ds(pl.multiple_of(base + coff, _CHUNK), _CHUNK)])

    return k(xi_flat, p_flat)


_BLK = 2048


def _finish(emb, x_t, w_full_t, bias):
    def body(emb_ref, x_ref, wn_ref, b_ref, o_ref):
        nsum = jnp.sum(x_ref[...] * wn_ref[...], axis=0)
        o_ref[...] = emb_ref[...] + nsum + b_ref[0, 0]

    return pl.pallas_call(
        body,
        grid=(_BATCH // _BLK,),
        in_specs=[
            pl.BlockSpec((_BLK,), lambda i: (i,)),
            pl.BlockSpec((_N_FEAT, _BLK), lambda i: (0, i)),
            pl.BlockSpec((_N_FEAT, 1), lambda i: (0, 0)),
            pl.BlockSpec((1, 1), lambda i: (0, 0)),
        ],
        out_specs=pl.BlockSpec((_BLK,), lambda i: (i,)),
        out_shape=jax.ShapeDtypeStruct((_BATCH,), jnp.float32),
    )(emb, x_t, w_full_t, bias)


def kernel(x, tables, W, b):
    x_t = jnp.transpose(x, (1, 0))
    tab_t = jnp.transpose(tables, (0, 2, 1))
    w_mat = W[0, : _N_CATS * _N_EMBED].reshape(_N_CATS, _N_EMBED)
    w_full_t = jnp.concatenate(
        [jnp.zeros((_N_CATS,), jnp.float32), W[0, _N_CATS * _N_EMBED:]]
    ).reshape(_N_FEAT, 1)
    bias = b.reshape(1, 1)

    xi_flat = _indices(x_t)
    p_flat = _project(tab_t, w_mat)
    emb = _sc_gather_sum(xi_flat, p_flat)
    return _finish(emb, x_t, w_full_t, bias).reshape(_BATCH, 1)


# speedup vs baseline: 84.8839x; 1.0482x over previous
import dataclasses
import functools

import jax
import jax.numpy as jnp
from jax import lax
from jax.experimental import pallas as pl
from jax.experimental.pallas import tpu as pltpu
from jax.experimental.pallas import tpu_sc as plsc

_N_CATS = 26
_VOCAB = 100000
_VPAD = 100352
_N_EMBED = 16
_N_NUM = 13
_N_FEAT = _N_CATS + _N_NUM
_BATCH = 16384

_NC = 2
_NS = 16
_NW = _NC * _NS
_SPW = _BATCH // _NW
_CHUNK = 128
_NCHUNK = _SPW // _CHUNK

_TPB = 2


def _indices(x_t):
    def body(x_ref, o_ref):
        a = pl.program_id(0)
        for r in range(8):
            o_ref[pl.ds(r * _BATCH, _BATCH)] = (
                x_ref[r].astype(jnp.int32) + (a * 8 + r) * _VPAD)

    return pl.pallas_call(
        body,
        grid=(_N_FEAT // 8,),
        in_specs=[pl.BlockSpec((8, _BATCH), lambda a: (a, 0))],
        out_specs=pl.BlockSpec((8 * _BATCH,), lambda a: (a,)),
        out_shape=jax.ShapeDtypeStruct((32 * _BATCH,), jnp.int32),
    )(x_t)


def _project(tab_t, w_mat):
    def body(tab_ref, w_ref, p_ref):
        for u in range(_TPB):
            p_ref[pl.ds(u * _VPAD, _VPAD)] = jnp.sum(
                tab_ref[u] * w_ref[u], axis=0)

    return pl.pallas_call(
        body,
        grid=(_N_CATS // _TPB,),
        in_specs=[
            pl.BlockSpec((_TPB, _N_EMBED, _VPAD), lambda t: (t, 0, 0)),
            pl.BlockSpec((_TPB, _N_EMBED, 1), lambda t: (t, 0, 0)),
        ],
        out_specs=pl.BlockSpec((_TPB * _VPAD,), lambda t: (t,)),
        out_shape=jax.ShapeDtypeStruct((_N_CATS * _VPAD,), jnp.float32),
    )(tab_t, w_mat[:, :, None])


def _sc_compiler_params():
    cp = pltpu.CompilerParams(use_tc_tiling_on_sc=False)
    if "needs_layout_passes" in pltpu.CompilerParams.__dataclass_fields__:
        cp = dataclasses.replace(cp, needs_layout_passes=False)
    return cp


def _sc_gather_sum(xi_flat, p_flat):
    mesh = plsc.VectorSubcoreMesh(core_axis_name="c", subcore_axis_name="s")

    @functools.partial(
        pl.kernel,
        out_type=jax.ShapeDtypeStruct((_BATCH,), jnp.float32),
        mesh=mesh,
        scratch_types=[
            pltpu.VMEM((_N_CATS * _SPW,), jnp.int32),
            pltpu.VMEM((_N_CATS * _CHUNK,), jnp.float32),
            pltpu.VMEM((_CHUNK,), jnp.float32),
            pltpu.SemaphoreType.DMA,
            pltpu.SemaphoreType.DMA,
        ],
        compiler_params=_sc_compiler_params(),
    )
    def k(xi_hbm, p_hbm, out_hbm, idx_v, vals_v, sum_v, isem, gsem):
        wid = lax.axis_index("s") * _NC + lax.axis_index("c")
        base = wid * _SPW

        idx_copies = [
            pltpu.async_copy(
                xi_hbm.at[pl.ds(pl.multiple_of(j * _BATCH + base, _SPW), _SPW)],
                idx_v.at[pl.ds(j * _SPW, _SPW)],
                isem,
            )
            for j in range(_N_CATS)
        ]
        for cp in idx_copies:
            cp.wait()

        @pl.loop(0, _NCHUNK)
        def _chunk(c):
            coff = c * _CHUNK

            gathers = [
                pltpu.async_copy(
                    p_hbm.at[idx_v.at[pl.ds(
                        pl.multiple_of(j * _SPW + coff, _CHUNK), _CHUNK)]],
                    vals_v.at[pl.ds(j * _CHUNK, _CHUNK)],
                    gsem,
                )
                for j in range(_N_CATS)
            ]
            for cp in gathers:
                cp.wait()

            for g in range(_CHUNK // 16):
                acc = vals_v[pl.ds(g * 16, 16)]
                for j in range(1, _N_CATS):
                    acc = acc + vals_v[pl.ds(j * _CHUNK + g * 16, 16)]
                sum_v[pl.ds(g * 16, 16)] = acc

            pltpu.sync_copy(
                sum_v,
                out_hbm.at[pl.ds(pl.multiple_of(base + coff, _CHUNK), _CHUNK)])

    return k(xi_flat, p_flat)


_BLK = 2048


def _finish(emb, x_t, w_full_t, bias):
    def body(emb_ref, x_ref, wn_ref, b_ref, o_ref):
        nsum = jnp.sum(x_ref[...] * wn_ref[...], axis=0)
        o_ref[...] = emb_ref[...] + nsum + b_ref[0, 0]

    return pl.pallas_call(
        body,
        grid=(_BATCH // _BLK,),
        in_specs=[
            pl.BlockSpec((_BLK,), lambda i: (i,)),
            pl.BlockSpec((_N_FEAT, _BLK), lambda i: (0, i)),
            pl.BlockSpec((_N_FEAT, 1), lambda i: (0, 0)),
            pl.BlockSpec((1, 1), lambda i: (0, 0)),
        ],
        out_specs=pl.BlockSpec((_BLK,), lambda i: (i,)),
        out_shape=jax.ShapeDtypeStruct((_BATCH,), jnp.float32),
    )(emb, x_t, w_full_t, bias)


def kernel(x, tables, W, b):
    x_t = jnp.transpose(x, (1, 0))
    tab_t = jnp.transpose(tables, (0, 2, 1))
    w_mat = W[0, : _N_CATS * _N_EMBED].reshape(_N_CATS, _N_EMBED)
    w_full_t = jnp.concatenate(
        [jnp.zeros((_N_CATS,), jnp.float32), W[0, _N_CATS * _N_EMBED:]]
    ).reshape(_N_FEAT, 1)
    bias = b.reshape(1, 1)

    xi_flat = _indices(x_t)
    p_flat = _project(tab_t, w_mat)
    emb = _sc_gather_sum(xi_flat, p_flat)
    return _finish(emb, x_t, w_full_t, bias).reshape(_BATCH, 1)


# speedup vs baseline: 88.4325x; 1.0418x over previous
import dataclasses
import functools

import jax
import jax.numpy as jnp
from jax import lax
from jax.experimental import pallas as pl
from jax.experimental.pallas import tpu as pltpu
from jax.experimental.pallas import tpu_sc as plsc

_N_CATS = 26
_VOCAB = 100000
_VPAD = 100352
_N_EMBED = 16
_N_NUM = 13
_N_FEAT = _N_CATS + _N_NUM
_BATCH = 16384

_NC = 2
_NS = 16
_NW = _NC * _NS
_SPW = _BATCH // _NW
_CHUNK = 128
_NCHUNK = _SPW // _CHUNK

_SPLIT = 14
_TPB = 2


def _indices(x_t):
    def body(x_ref, o_ref):
        for t in range(_N_CATS):
            rel = t if t < _SPLIT else t - _SPLIT
            o_ref[pl.ds(t * _BATCH, _BATCH)] = (
                x_ref[t].astype(jnp.int32) + rel * _VPAD)

    return pl.pallas_call(
        body,
        grid=(1,),
        in_specs=[pl.BlockSpec((32, _BATCH), lambda i: (0, 0))],
        out_specs=pl.BlockSpec((_N_CATS * _BATCH,), lambda i: (0,)),
        out_shape=jax.ShapeDtypeStruct((_N_CATS * _BATCH,), jnp.int32),
    )(x_t)


def _project(tab_t, w_col, t0, tn):
    def body(tab_ref, w_ref, p_ref):
        for u in range(_TPB):
            p_ref[pl.ds(u * _VPAD, _VPAD)] = jnp.sum(
                tab_ref[u] * w_ref[u], axis=0)

    return pl.pallas_call(
        body,
        grid=(tn // _TPB,),
        in_specs=[
            pl.BlockSpec((_TPB, _N_EMBED, _VPAD),
                         lambda t: (t0 // _TPB + t, 0, 0)),
            pl.BlockSpec((_TPB, _N_EMBED, 1), lambda t: (t0 // _TPB + t, 0, 0)),
        ],
        out_specs=pl.BlockSpec((_TPB * _VPAD,), lambda t: (t,)),
        out_shape=jax.ShapeDtypeStruct((tn * _VPAD,), jnp.float32),
    )(tab_t, w_col)


def _sc_compiler_params():
    cp = pltpu.CompilerParams(use_tc_tiling_on_sc=False)
    if "needs_layout_passes" in pltpu.CompilerParams.__dataclass_fields__:
        cp = dataclasses.replace(cp, needs_layout_passes=False)
    return cp


def _sc_gather_sum(xi_flat, p_flat, t0, tn):
    mesh = plsc.VectorSubcoreMesh(core_axis_name="c", subcore_axis_name="s")

    @functools.partial(
        pl.kernel,
        out_type=jax.ShapeDtypeStruct((_BATCH,), jnp.float32),
        mesh=mesh,
        scratch_types=[
            pltpu.VMEM((tn * _SPW,), jnp.int32),
            pltpu.VMEM((tn * _CHUNK,), jnp.float32),
            pltpu.VMEM((_CHUNK,), jnp.float32),
            pltpu.SemaphoreType.DMA,
            pltpu.SemaphoreType.DMA,
        ],
        compiler_params=_sc_compiler_params(),
    )
    def k(xi_hbm, p_hbm, out_hbm, idx_v, vals_v, sum_v, isem, gsem):
        wid = lax.axis_index("s") * _NC + lax.axis_index("c")
        base = wid * _SPW

        idx_copies = [
            pltpu.async_copy(
                xi_hbm.at[pl.ds(
                    pl.multiple_of((t0 + j) * _BATCH + base, _SPW), _SPW)],
                idx_v.at[pl.ds(j * _SPW, _SPW)],
                isem,
            )
            for j in range(tn)
        ]
        for cp in idx_copies:
            cp.wait()

        @pl.loop(0, _NCHUNK)
        def _chunk(c):
            coff = c * _CHUNK

            gathers = [
                pltpu.async_copy(
                    p_hbm.at[idx_v.at[pl.ds(
                        pl.multiple_of(j * _SPW + coff, _CHUNK), _CHUNK)]],
                    vals_v.at[pl.ds(j * _CHUNK, _CHUNK)],
                    gsem,
                )
                for j in range(tn)
            ]
            for cp in gathers:
                cp.wait()

            for g in range(_CHUNK // 16):
                acc = vals_v[pl.ds(g * 16, 16)]
                for j in range(1, tn):
                    acc = acc + vals_v[pl.ds(j * _CHUNK + g * 16, 16)]
                sum_v[pl.ds(g * 16, 16)] = acc

            pltpu.sync_copy(
                sum_v,
                out_hbm.at[pl.ds(pl.multiple_of(base + coff, _CHUNK), _CHUNK)])

    return k(xi_flat, p_flat)


_BLK = 4096


def _numeric(x_t, w_full_t, bias):
    def body(x_ref, wn_ref, b_ref, o_ref):
        o_ref[...] = jnp.sum(x_ref[...] * wn_ref[...], axis=0) + b_ref[0, 0]

    return pl.pallas_call(
        body,
        grid=(_BATCH // _BLK,),
        in_specs=[
            pl.BlockSpec((_N_FEAT, _BLK), lambda i: (0, i)),
            pl.BlockSpec((_N_FEAT, 1), lambda i: (0, 0)),
            pl.BlockSpec((1, 1), lambda i: (0, 0)),
        ],
        out_specs=pl.BlockSpec((_BLK,), lambda i: (i,)),
        out_shape=jax.ShapeDtypeStruct((_BATCH,), jnp.float32),
    )(x_t, w_full_t, bias)


def _combine(emb_a, emb_b, num):
    def body(a_ref, b_ref, n_ref, o_ref):
        o_ref[...] = a_ref[...] + b_ref[...] + n_ref[...]

    return pl.pallas_call(
        body,
        grid=(_BATCH // _BLK,),
        in_specs=[pl.BlockSpec((_BLK,), lambda i: (i,))] * 3,
        out_specs=pl.BlockSpec((_BLK,), lambda i: (i,)),
        out_shape=jax.ShapeDtypeStruct((_BATCH,), jnp.float32),
    )(emb_a, emb_b, num)


def kernel(x, tables, W, b):
    x_t = jnp.transpose(x, (1, 0))
    tab_t = jnp.transpose(tables, (0, 2, 1))
    w_col = W[0, : _N_CATS * _N_EMBED].reshape(_N_CATS, _N_EMBED)[:, :, None]
    w_full_t = jnp.concatenate(
        [jnp.zeros((_N_CATS,), jnp.float32), W[0, _N_CATS * _N_EMBED:]]
    ).reshape(_N_FEAT, 1)
    bias = b.reshape(1, 1)

    xi_flat = _indices(x_t)
    p_a = _project(tab_t, w_col, 0, _SPLIT)
    emb_a = _sc_gather_sum(xi_flat, p_a, 0, _SPLIT)
    p_b = _project(tab_t, w_col, _SPLIT, _N_CATS - _SPLIT)
    emb_b = _sc_gather_sum(xi_flat, p_b, _SPLIT, _N_CATS - _SPLIT)
    num = _numeric(x_t, w_full_t, bias)
    return _combine(emb_a, emb_b, num).reshape(_BATCH, 1)


# speedup vs baseline: 88.6840x; 1.0028x over previous
import dataclasses
import functools

import jax
import jax.numpy as jnp
from jax import lax
from jax.experimental import pallas as pl
from jax.experimental.pallas import tpu as pltpu
from jax.experimental.pallas import tpu_sc as plsc

_N_CATS = 26
_VOCAB = 100000
_VPAD = 100352
_N_EMBED = 16
_N_NUM = 13
_N_FEAT = _N_CATS + _N_NUM
_BATCH = 16384

_NC = 2
_NS = 16
_NW = _NC * _NS
_SPW = _BATCH // _NW
_CHUNK = 256
_NCHUNK = _SPW // _CHUNK

_SPLIT = 14
_TPB = 2


def _indices(x_t):
    def body(x_ref, o_ref):
        for t in range(_N_CATS):
            rel = t if t < _SPLIT else t - _SPLIT
            o_ref[pl.ds(t * _BATCH, _BATCH)] = (
                x_ref[t].astype(jnp.int32) + rel * _VPAD)

    return pl.pallas_call(
        body,
        grid=(1,),
        in_specs=[pl.BlockSpec((32, _BATCH), lambda i: (0, 0))],
        out_specs=pl.BlockSpec((_N_CATS * _BATCH,), lambda i: (0,)),
        out_shape=jax.ShapeDtypeStruct((_N_CATS * _BATCH,), jnp.int32),
    )(x_t)


def _project(tab_t, w_col, t0, tn):
    def body(tab_ref, w_ref, p_ref):
        for u in range(_TPB):
            p_ref[pl.ds(u * _VPAD, _VPAD)] = jnp.sum(
                tab_ref[u] * w_ref[u], axis=0)

    return pl.pallas_call(
        body,
        grid=(tn // _TPB,),
        in_specs=[
            pl.BlockSpec((_TPB, _N_EMBED, _VPAD),
                         lambda t: (t0 // _TPB + t, 0, 0)),
            pl.BlockSpec((_TPB, _N_EMBED, 1), lambda t: (t0 // _TPB + t, 0, 0)),
        ],
        out_specs=pl.BlockSpec((_TPB * _VPAD,), lambda t: (t,)),
        out_shape=jax.ShapeDtypeStruct((tn * _VPAD,), jnp.float32),
    )(tab_t, w_col)


def _sc_compiler_params():
    cp = pltpu.CompilerParams(use_tc_tiling_on_sc=False)
    if "needs_layout_passes" in pltpu.CompilerParams.__dataclass_fields__:
        cp = dataclasses.replace(cp, needs_layout_passes=False)
    return cp


def _sc_gather_sum(xi_flat, p_flat, t0, tn):
    mesh = plsc.VectorSubcoreMesh(core_axis_name="c", subcore_axis_name="s")

    @functools.partial(
        pl.kernel,
        out_type=jax.ShapeDtypeStruct((_BATCH,), jnp.float32),
        mesh=mesh,
        scratch_types=[
            pltpu.VMEM((tn * _SPW,), jnp.int32),
            pltpu.VMEM((tn * _CHUNK,), jnp.float32),
            pltpu.VMEM((_CHUNK,), jnp.float32),
            pltpu.SemaphoreType.DMA,
            pltpu.SemaphoreType.DMA,
        ],
        compiler_params=_sc_compiler_params(),
    )
    def k(xi_hbm, p_hbm, out_hbm, idx_v, vals_v, sum_v, isem, gsem):
        wid = lax.axis_index("s") * _NC + lax.axis_index("c")
        base = wid * _SPW

        idx_copies = [
            pltpu.async_copy(
                xi_hbm.at[pl.ds(
                    pl.multiple_of((t0 + j) * _BATCH + base, _SPW), _SPW)],
                idx_v.at[pl.ds(j * _SPW, _SPW)],
                isem,
            )
            for j in range(tn)
        ]
        for cp in idx_copies:
            cp.wait()

        @pl.loop(0, _NCHUNK)
        def _chunk(c):
            coff = c * _CHUNK

            gathers = [
                pltpu.async_copy(
                    p_hbm.at[idx_v.at[pl.ds(
                        pl.multiple_of(j * _SPW + coff, _CHUNK), _CHUNK)]],
                    vals_v.at[pl.ds(j * _CHUNK, _CHUNK)],
                    gsem,
                )
                for j in range(tn)
            ]
            for cp in gathers:
                cp.wait()

            for g in range(_CHUNK // 16):
                acc = vals_v[pl.ds(g * 16, 16)]
                for j in range(1, tn):
                    acc = acc + vals_v[pl.ds(j * _CHUNK + g * 16, 16)]
                sum_v[pl.ds(g * 16, 16)] = acc

            pltpu.sync_copy(
                sum_v,
                out_hbm.at[pl.ds(pl.multiple_of(base + coff, _CHUNK), _CHUNK)])

    return k(xi_flat, p_flat)


_BLK = 4096


def _numeric(x_t, w_full_t, bias):
    def body(x_ref, wn_ref, b_ref, o_ref):
        o_ref[...] = jnp.sum(x_ref[...] * wn_ref[...], axis=0) + b_ref[0, 0]

    return pl.pallas_call(
        body,
        grid=(_BATCH // _BLK,),
        in_specs=[
            pl.BlockSpec((_N_FEAT, _BLK), lambda i: (0, i)),
            pl.BlockSpec((_N_FEAT, 1), lambda i: (0, 0)),
            pl.BlockSpec((1, 1), lambda i: (0, 0)),
        ],
        out_specs=pl.BlockSpec((_BLK,), lambda i: (i,)),
        out_shape=jax.ShapeDtypeStruct((_BATCH,), jnp.float32),
    )(x_t, w_full_t, bias)


def _combine(emb_a, emb_b, num):
    def body(a_ref, b_ref, n_ref, o_ref):
        o_ref[...] = a_ref[...] + b_ref[...] + n_ref[...]

    return pl.pallas_call(
        body,
        grid=(_BATCH // _BLK,),
        in_specs=[pl.BlockSpec((_BLK,), lambda i: (i,))] * 3,
        out_specs=pl.BlockSpec((_BLK,), lambda i: (i,)),
        out_shape=jax.ShapeDtypeStruct((_BATCH,), jnp.float32),
    )(emb_a, emb_b, num)


def kernel(x, tables, W, b):
    x_t = jnp.transpose(x, (1, 0))
    tab_t = jnp.transpose(tables, (0, 2, 1))
    w_col = W[0, : _N_CATS * _N_EMBED].reshape(_N_CATS, _N_EMBED)[:, :, None]
    w_full_t = jnp.concatenate(
        [jnp.zeros((_N_CATS,), jnp.float32), W[0, _N_CATS * _N_EMBED:]]
    ).reshape(_N_FEAT, 1)
    bias = b.reshape(1, 1)

    xi_flat = _indices(x_t)
    p_a = _project(tab_t, w_col, 0, _SPLIT)
    emb_a = _sc_gather_sum(xi_flat, p_a, 0, _SPLIT)
    p_b = _project(tab_t, w_col, _SPLIT, _N_CATS - _SPLIT)
    emb_b = _sc_gather_sum(xi_flat, p_b, _SPLIT, _N_CATS - _SPLIT)
    num = _numeric(x_t, w_full_t, bias)
    return _combine(emb_a, emb_b, num).reshape(_BATCH, 1)


# speedup vs baseline: 88.7805x; 1.0011x over previous
import dataclasses
import functools

import jax
import jax.numpy as jnp
from jax import lax
from jax.experimental import pallas as pl
from jax.experimental.pallas import tpu as pltpu
from jax.experimental.pallas import tpu_sc as plsc

_N_CATS = 26
_VOCAB = 100000
_VPAD = 100352
_N_EMBED = 16
_N_NUM = 13
_N_FEAT = _N_CATS + _N_NUM
_BATCH = 16384

_NC = 2
_NS = 16
_NW = _NC * _NS
_SPW = _BATCH // _NW
_CHUNK = 256
_NCHUNK = _SPW // _CHUNK

_SPLIT = 14
_TPB = 2


def _indices(x_t):
    def body(x_ref, o_ref):
        for t in range(_N_CATS):
            rel = t if t < _SPLIT else t - _SPLIT
            o_ref[pl.ds(t * _BATCH, _BATCH)] = (
                x_ref[t].astype(jnp.int32) + rel * _VPAD)

    return pl.pallas_call(
        body,
        grid=(1,),
        in_specs=[pl.BlockSpec((32, _BATCH), lambda i: (0, 0))],
        out_specs=pl.BlockSpec((_N_CATS * _BATCH,), lambda i: (0,)),
        out_shape=jax.ShapeDtypeStruct((_N_CATS * _BATCH,), jnp.int32),
    )(x_t)


def _project(tab_t, w_col, t0, tn):
    def body(tab_ref, w_ref, p_ref):
        for u in range(_TPB):
            p_ref[pl.ds(u * _VPAD, _VPAD)] = jnp.sum(
                tab_ref[u] * w_ref[u], axis=0)

    return pl.pallas_call(
        body,
        grid=(tn // _TPB,),
        in_specs=[
            pl.BlockSpec((_TPB, _N_EMBED, _VPAD),
                         lambda t: (t0 // _TPB + t, 0, 0)),
            pl.BlockSpec((_TPB, _N_EMBED, 1), lambda t: (t0 // _TPB + t, 0, 0)),
        ],
        out_specs=pl.BlockSpec((_TPB * _VPAD,), lambda t: (t,)),
        out_shape=jax.ShapeDtypeStruct((tn * _VPAD,), jnp.float32),
    )(tab_t, w_col)


def _sc_compiler_params():
    cp = pltpu.CompilerParams(use_tc_tiling_on_sc=False)
    if "needs_layout_passes" in pltpu.CompilerParams.__dataclass_fields__:
        cp = dataclasses.replace(cp, needs_layout_passes=False)
    return cp


def _sc_gather_sum(xi_flat, p_flat, t0, tn):
    mesh = plsc.VectorSubcoreMesh(core_axis_name="c", subcore_axis_name="s")

    @functools.partial(
        pl.kernel,
        out_type=jax.ShapeDtypeStruct((_BATCH,), jnp.float32),
        mesh=mesh,
        scratch_types=[
            pltpu.VMEM((tn * _SPW,), jnp.int32),
            pltpu.VMEM((tn * _CHUNK,), jnp.float32),
            pltpu.VMEM((_CHUNK,), jnp.float32),
            pltpu.SemaphoreType.DMA,
            pltpu.SemaphoreType.DMA,
        ],
        compiler_params=_sc_compiler_params(),
    )
    def k(xi_hbm, p_hbm, out_hbm, idx_v, vals_v, sum_v, isem, gsem):
        wid = lax.axis_index("s") * _NC + lax.axis_index("c")
        base = wid * _SPW

        idx_copies = [
            pltpu.async_copy(
                xi_hbm.at[pl.ds(
                    pl.multiple_of((t0 + j) * _BATCH + base, _SPW), _SPW)],
                idx_v.at[pl.ds(j * _SPW, _SPW)],
                isem,
            )
            for j in range(tn)
        ]
        for cp in idx_copies:
            cp.wait()

        @pl.loop(0, _NCHUNK)
        def _chunk(c):
            coff = c * _CHUNK

            gathers = [
                pltpu.async_copy(
                    p_hbm.at[idx_v.at[pl.ds(
                        pl.multiple_of(j * _SPW + coff, _CHUNK), _CHUNK)]],
                    vals_v.at[pl.ds(j * _CHUNK, _CHUNK)],
                    gsem,
                )
                for j in range(tn)
            ]
            for cp in gathers:
                cp.wait()

            for g in range(_CHUNK // 16):
                acc = vals_v[pl.ds(g * 16, 16)]
                for j in range(1, tn):
                    acc = acc + vals_v[pl.ds(j * _CHUNK + g * 16, 16)]
                sum_v[pl.ds(g * 16, 16)] = acc

            pltpu.sync_copy(
                sum_v,
                out_hbm.at[pl.ds(pl.multiple_of(base + coff, _CHUNK), _CHUNK)])

    return k(xi_flat, p_flat)


_BLK = 4096


def _numeric(x_t, w_full_t, bias):
    def body(x_ref, wn_ref, b_ref, o_ref):
        xb = x_ref[...].astype(jnp.bfloat16).astype(jnp.float32)
        wb = wn_ref[...].astype(jnp.bfloat16).astype(jnp.float32)
        o_ref[...] = jnp.sum(xb * wb, axis=0) + b_ref[0, 0]

    return pl.pallas_call(
        body,
        grid=(_BATCH // _BLK,),
        in_specs=[
            pl.BlockSpec((_N_FEAT, _BLK), lambda i: (0, i)),
            pl.BlockSpec((_N_FEAT, 1), lambda i: (0, 0)),
            pl.BlockSpec((1, 1), lambda i: (0, 0)),
        ],
        out_specs=pl.BlockSpec((_BLK,), lambda i: (i,)),
        out_shape=jax.ShapeDtypeStruct((_BATCH,), jnp.float32),
    )(x_t, w_full_t, bias)


def _combine(emb_a, emb_b, num):
    def body(a_ref, b_ref, n_ref, o_ref):
        o_ref[...] = a_ref[...] + b_ref[...] + n_ref[...]

    return pl.pallas_call(
        body,
        grid=(_BATCH // _BLK,),
        in_specs=[pl.BlockSpec((_BLK,), lambda i: (i,))] * 3,
        out_specs=pl.BlockSpec((_BLK,), lambda i: (i,)),
        out_shape=jax.ShapeDtypeStruct((_BATCH,), jnp.float32),
    )(emb_a, emb_b, num)


def kernel(x, tables, W, b):
    x_t = jnp.transpose(x, (1, 0))
    tab_t = jnp.transpose(tables, (0, 2, 1))
    w_col = W[0, : _N_CATS * _N_EMBED].reshape(_N_CATS, _N_EMBED)[:, :, None]
    w_full_t = jnp.concatenate(
        [jnp.zeros((_N_CATS,), jnp.float32), W[0, _N_CATS * _N_EMBED:]]
    ).reshape(_N_FEAT, 1)
    bias = b.reshape(1, 1)

    xi_flat = _indices(x_t)
    p_a = _project(tab_t, w_col, 0, _SPLIT)
    emb_a = _sc_gather_sum(xi_flat, p_a, 0, _SPLIT)
    p_b = _project(tab_t, w_col, _SPLIT, _N_CATS - _SPLIT)
    emb_b = _sc_gather_sum(xi_flat, p_b, _SPLIT, _N_CATS - _SPLIT)
    num = _numeric(x_t, w_full_t, bias)
    return _combine(emb_a, emb_b, num).reshape(_BATCH, 1)
